# baseline (speedup 1.0000x reference)
import jax
import jax.numpy as jnp
from jax import lax
from jax.experimental import pallas as pl
from jax.experimental.pallas import tpu as pltpu
from jax.experimental.pallas import tpu_sc as plsc

B, L, D = 1024, 200, 64
NC, NS = 2, 16
NW = NC * NS
CHB = 128
NGB = B // CHB
NQ = NW // NGB
NT = L // NQ


def _body(idxw_hbm, parw_hbm, posq_hbm, tok2_hbm, out_hbm,
          idxs, pars, posb0, posb1, gbuf0, gbuf1, outb0, outb1,
          gsem0, gsem1, psem0, psem1, osem0, osem1):
    c = lax.axis_index("c")
    s = lax.axis_index("s")
    w = s * NC + c
    q = w // NGB
    b0 = (w % NGB) * CHB

    pltpu.sync_copy(idxw_hbm.at[w], idxs)
    pltpu.sync_copy(parw_hbm.at[w], pars)

    gbufs = (gbuf0, gbuf1)
    posbs = (posb0, posb1)
    outbs = (outb0, outb1)
    gsems = (gsem0, gsem1)
    psems = (psem0, psem1)
    osems = (osem0, osem1)

    jv = [jnp.int32(g * 16) + lax.iota(jnp.int32, 16) for g in range(8)]

    def gather(t):
        return pltpu.async_copy(tok2_hbm.at[idxs.at[t]],
                                gbufs[t % 2], gsems[t % 2])

    def posdma(t):
        return pltpu.async_copy(posq_hbm.at[q, t], posbs[t % 2],
                                psems[t % 2])

    def out_start(t):
        return pltpu.async_copy(
            outbs[t % 2],
            out_hbm.at[NQ * t + q, :, pl.ds(b0, CHB)],
            osems[t % 2])

    def compute(t):
        gb = gbufs[t % 2]
        pb = posbs[t % 2]
        ob = outbs[t % 2]
        cvb = [pars[t, pl.ds(g * 16, 16)] << 6 for g in range(8)]

        def dstep(d, carry):
            posd = pb[pl.ds(d * 16, 16)]
            for g in range(8):
                v = plsc.load_gather(gb, [jv[g], cvb[g] + d])
                ob[d, pl.ds(g * 16, 16)] = v + posd
            return carry

        lax.fori_loop(0, D, dstep, 0)

    cps_g = [None] * NT
    cps_p = [None] * NT
    cps_o = [None] * NT
    cps_g[0] = gather(0)
    cps_p[0] = posdma(0)
    for t in range(NT):
        if t + 1 < NT:
            cps_g[t + 1] = gather(t + 1)
            cps_p[t + 1] = posdma(t + 1)
        cps_g[t].wait()
        cps_p[t].wait()
        if t - 2 >= 0:
            cps_o[t - 2].wait()
        compute(t)
        cps_o[t] = out_start(t)
    cps_o[NT - 2].wait()
    cps_o[NT - 1].wait()


def kernel(x, tok_table, pos_table):
    xt = x.T.astype(jnp.int32)
    idx2 = (xt >> 1).reshape(NT, NQ, NGB, CHB)
    idxw = idx2.transpose(1, 2, 0, 3).reshape(NW, NT, CHB)
    par2 = (xt & 1).reshape(NT, NQ, NGB, CHB)
    parw = par2.transpose(1, 2, 0, 3).reshape(NW, NT, CHB)
    pos_rb = jnp.repeat(pos_table, 16, axis=1)
    posq = pos_rb.reshape(NT, NQ, 16 * D).transpose(1, 0, 2)
    tok2 = tok_table.reshape(500000, 2 * D)

    mesh = plsc.VectorSubcoreMesh(core_axis_name="c", subcore_axis_name="s")
    run = pl.kernel(
        _body,
        out_type=jax.ShapeDtypeStruct((L, D, B), jnp.float32),
        mesh=mesh,
        compiler_params=pltpu.CompilerParams(needs_layout_passes=False),
        scratch_types=[
            pltpu.VMEM((NT, CHB), jnp.int32),
            pltpu.VMEM((NT, CHB), jnp.int32),
            pltpu.VMEM((16 * D,), jnp.float32),
            pltpu.VMEM((16 * D,), jnp.float32),
            pltpu.VMEM((CHB, 2 * D), jnp.float32),
            pltpu.VMEM((CHB, 2 * D), jnp.float32),
            pltpu.VMEM((D, CHB), jnp.float32),
            pltpu.VMEM((D, CHB), jnp.float32),
            pltpu.SemaphoreType.DMA,
            pltpu.SemaphoreType.DMA,
            pltpu.SemaphoreType.DMA,
            pltpu.SemaphoreType.DMA,
            pltpu.SemaphoreType.DMA,
            pltpu.SemaphoreType.DMA,
        ],
    )
    out_t = run(idxw, parw, posq, tok2)
    return out_t.transpose(2, 0, 1)


# speedup vs baseline: 1.2016x; 1.2016x over previous
import jax
import jax.numpy as jnp
from jax import lax
from jax.experimental import pallas as pl
from jax.experimental.pallas import tpu as pltpu
from jax.experimental.pallas import tpu_sc as plsc

B, L, D = 1024, 200, 64
NC, NS = 2, 16
NW = NC * NS
ROWS_PER_W = B * L // NW
CH = 128
NCH = ROWS_PER_W // CH


def _body(x_hbm, tok_hbm, pos2_hbm, out_hbm,
          idx_v, pos_v, buf0, buf1, gsem0, gsem1, osem0, osem1):
    c = lax.axis_index("c")
    s = lax.axis_index("s")
    w = s * NC + c

    pltpu.sync_copy(x_hbm.at[w], idx_v)
    pltpu.sync_copy(pos2_hbm, pos_v)

    bufs = (buf0, buf1)
    gsems = (gsem0, gsem1)
    osems = (osem0, osem1)

    def gather(cc):
        return pltpu.async_copy(tok_hbm.at[idx_v.at[cc]],
                                bufs[cc % 2], gsems[cc % 2])

    def out_start(cc):
        return pltpu.async_copy(
            bufs[cc % 2],
            out_hbm.at[pl.ds(w * ROWS_PER_W + cc * CH, CH)],
            osems[cc % 2])

    def add_pos(cc):
        b = bufs[cc % 2]
        p = (cc * CH) % L

        def add4(k, carry):
            for j in range(4):
                r = 4 * k + j
                for dd in range(0, D, 16):
                    b[r, pl.ds(dd, 16)] = (b[r, pl.ds(dd, 16)]
                                           + pos_v[p + r, pl.ds(dd, 16)])
            return carry

        lax.fori_loop(0, CH // 4, add4, 0)

    cps_g = [None] * NCH
    cps_o = [None] * NCH
    cps_g[0] = gather(0)
    for cc in range(NCH):
        if cc + 1 < NCH:
            if cc - 1 >= 0:
                cps_o[cc - 1].wait()
            cps_g[cc + 1] = gather(cc + 1)
        cps_g[cc].wait()
        add_pos(cc)
        cps_o[cc] = out_start(cc)
    cps_o[NCH - 2].wait()
    cps_o[NCH - 1].wait()


def kernel(x, tok_table, pos_table):
    x3 = x.reshape(NW, NCH, CH).astype(jnp.int32)
    pos2 = jnp.concatenate([pos_table, pos_table], axis=0)
    mesh = plsc.VectorSubcoreMesh(core_axis_name="c", subcore_axis_name="s")
    run = pl.kernel(
        _body,
        out_type=jax.ShapeDtypeStruct((B * L, D), jnp.float32),
        mesh=mesh,
        compiler_params=pltpu.CompilerParams(use_tc_tiling_on_sc=False),
        scratch_types=[
            pltpu.VMEM((NCH, CH), jnp.int32),
            pltpu.VMEM((2 * L, D), jnp.float32),
            pltpu.VMEM((CH, D), jnp.float32),
            pltpu.VMEM((CH, D), jnp.float32),
            pltpu.SemaphoreType.DMA,
            pltpu.SemaphoreType.DMA,
            pltpu.SemaphoreType.DMA,
            pltpu.SemaphoreType.DMA,
        ],
    )
    out = run(x3, tok_table, pos2)
    return out.reshape(B, L, D)


# speedup vs baseline: 1.2114x; 1.0081x over previous
import jax
import jax.numpy as jnp
from jax import lax
from jax.experimental import pallas as pl
from jax.experimental.pallas import tpu as pltpu
from jax.experimental.pallas import tpu_sc as plsc

B, L, D = 1024, 200, 64
NC, NS = 2, 16
NW = NC * NS
ROWS_PER_W = B * L // NW
CH = 128
NCH = ROWS_PER_W // CH
NBUF = 4


def _body(x_hbm, tok_hbm, pos2_hbm, out_hbm,
          idx_v, pos_v, buf0, buf1, buf2, buf3,
          gsem0, gsem1, gsem2, gsem3, osem0, osem1, osem2, osem3):
    c = lax.axis_index("c")
    s = lax.axis_index("s")
    w = s * NC + c

    pltpu.sync_copy(x_hbm.at[w], idx_v)
    pltpu.sync_copy(pos2_hbm, pos_v)

    bufs = (buf0, buf1, buf2, buf3)
    gsems = (gsem0, gsem1, gsem2, gsem3)
    osems = (osem0, osem1, osem2, osem3)

    def gather(cc):
        return pltpu.async_copy(tok_hbm.at[idx_v.at[cc]],
                                bufs[cc % NBUF], gsems[cc % NBUF])

    def out_start(cc):
        return pltpu.async_copy(
            bufs[cc % NBUF],
            out_hbm.at[pl.ds(w * ROWS_PER_W + cc * CH, CH)],
            osems[cc % NBUF])

    def add_pos(cc):
        b = bufs[cc % NBUF]
        p = (cc * CH) % L

        def add4(k, carry):
            for j in range(4):
                r = 4 * k + j
                for dd in range(0, D, 16):
                    b[r, pl.ds(dd, 16)] = (b[r, pl.ds(dd, 16)]
                                           + pos_v[p + r, pl.ds(dd, 16)])
            return carry

        lax.fori_loop(0, CH // 4, add4, 0)

    cps_g = [None] * NCH
    cps_o = [None] * NCH
    for cc in range(min(NBUF, NCH)):
        cps_g[cc] = gather(cc)
    for cc in range(NCH):
        cps_g[cc].wait()
        add_pos(cc)
        cps_o[cc] = out_start(cc)
        if cc + NBUF < NCH:
            cps_o[cc].wait()
            cps_g[cc + NBUF] = gather(cc + NBUF)
    for cc in range(max(0, NCH - NBUF), NCH):
        cps_o[cc].wait()


def kernel(x, tok_table, pos_table):
    x3 = x.reshape(NW, NCH, CH).astype(jnp.int32)
    pos2 = jnp.concatenate([pos_table, pos_table], axis=0)
    mesh = plsc.VectorSubcoreMesh(core_axis_name="c", subcore_axis_name="s")
    run = pl.kernel(
        _body,
        out_type=jax.ShapeDtypeStruct((B * L, D), jnp.float32),
        mesh=mesh,
        compiler_params=pltpu.CompilerParams(use_tc_tiling_on_sc=False),
        scratch_types=[
            pltpu.VMEM((NCH, CH), jnp.int32),
            pltpu.VMEM((2 * L, D), jnp.float32),
            pltpu.VMEM((CH, D), jnp.float32),
            pltpu.VMEM((CH, D), jnp.float32),
            pltpu.VMEM((CH, D), jnp.float32),
            pltpu.VMEM((CH, D), jnp.float32),
            pltpu.SemaphoreType.DMA,
            pltpu.SemaphoreType.DMA,
            pltpu.SemaphoreType.DMA,
            pltpu.SemaphoreType.DMA,
            pltpu.SemaphoreType.DMA,
            pltpu.SemaphoreType.DMA,
            pltpu.SemaphoreType.DMA,
            pltpu.SemaphoreType.DMA,
        ],
    )
    out = run(x3, tok_table, pos2)
    return out.reshape(B, L, D)
